# baseline (speedup 1.0000x reference)
import functools

import jax
import jax.numpy as jnp
from jax.experimental import pallas as pl
from jax.experimental.pallas import tpu as pltpu
from jax.experimental.pallas import tpu_sc as plsc

DMODEL = 1024
N_EXPERTS = 8
EXPERT_SIZE = 512
T = 2048

BLK = 256
NBLK = T * 2 // BLK + N_EXPERTS
NSLOT = NBLK * BLK
WPAD = 128

SC_CORES = 2
SC_SUBCORES = 16
SC_WORKERS = SC_CORES * SC_SUBCORES
TOK_PER_W = T // SC_WORKERS
CHUNK = 16


def _router_kernel(x_ref, sel_ref, pos0_ref, pos1_ref, w0_ref, w1_ref,
                   be_ref, oe_ref, na_ref):
    x = x_ref[...]
    logits = jax.lax.dot_general(
        x, sel_ref[...], (((1,), (1,)), ((), ())),
        preferred_element_type=jnp.float32)
    idx8 = jax.lax.broadcasted_iota(jnp.int32, logits.shape, 1)
    m1 = jnp.max(logits, axis=1, keepdims=True)
    i1 = jnp.min(jnp.where(logits == m1, idx8, N_EXPERTS), axis=1,
                 keepdims=True)
    oh1 = idx8 == i1
    rest = jnp.where(oh1, -jnp.inf, logits)
    m2 = jnp.max(rest, axis=1, keepdims=True)
    i2 = jnp.min(jnp.where(rest == m2, idx8, N_EXPERTS), axis=1,
                 keepdims=True)
    oh2 = idx8 == i2
    sig = jax.nn.sigmoid(logits)
    v0 = jnp.sum(jnp.where(oh1, sig, 0.0), axis=1, keepdims=True)
    v1 = jnp.sum(jnp.where(oh2, sig, 0.0), axis=1, keepdims=True)

    cnt = (oh1 | oh2).astype(jnp.float32)
    inc = cnt
    sh = 1
    while sh < T:
        inc = inc + jnp.concatenate(
            [jnp.zeros((sh, N_EXPERTS), jnp.float32), inc[:T - sh]], axis=0)
        sh *= 2
    rank = inc - cnt
    tot = inc[T - 1:T, :]
    padded = jnp.floor((tot + (BLK - 1)) / BLK) * BLK
    u_r = jax.lax.broadcasted_iota(jnp.int32, (N_EXPERTS, N_EXPERTS), 0)
    u_c = jax.lax.broadcasted_iota(jnp.int32, (N_EXPERTS, N_EXPERTS), 1)
    ustri = (u_r < u_c).astype(jnp.float32)
    off = jax.lax.dot_general(
        padded, ustri, (((1,), (0,)), ((), ())),
        preferred_element_type=jnp.float32)

    pos_base = off + rank
    p0 = jnp.sum(jnp.where(oh1, pos_base, 0.0), axis=1, keepdims=True)
    p1 = jnp.sum(jnp.where(oh2, pos_base, 0.0), axis=1, keepdims=True)

    pos0_ref[...] = jnp.broadcast_to(p0, (T, N_EXPERTS)).astype(jnp.int32)
    pos1_ref[...] = jnp.broadcast_to(p1, (T, N_EXPERTS)).astype(jnp.int32)
    w0_ref[...] = jnp.broadcast_to(v0, (T, WPAD))
    w1_ref[...] = jnp.broadcast_to(v1, (T, WPAD))

    total = jnp.sum(padded, axis=1, keepdims=True)
    na_ref[...] = (total / BLK).astype(jnp.int32)
    starts = (jax.lax.broadcasted_iota(jnp.int32, (NBLK, 1), 0) * BLK
              ).astype(jnp.float32)
    off_next = off + padded
    blk_raw = jnp.minimum(
        jnp.sum((starts >= off_next).astype(jnp.float32), axis=1,
                keepdims=True), float(N_EXPERTS - 1))
    blk_i = blk_raw.astype(jnp.int32)
    lane8 = jax.lax.broadcasted_iota(jnp.int32, (NBLK, N_EXPERTS), 1)
    onehot = blk_i == lane8
    oe_col = jnp.sum(jnp.where(onehot, off + tot, 0.0), axis=1, keepdims=True)
    be_ref[...] = blk_i
    oe_ref[...] = oe_col.astype(jnp.int32)


def _run_router(x, expert_sel):
    return pl.pallas_call(
        _router_kernel,
        out_shape=[
            jax.ShapeDtypeStruct((T, N_EXPERTS), jnp.int32),
            jax.ShapeDtypeStruct((T, N_EXPERTS), jnp.int32),
            jax.ShapeDtypeStruct((T, WPAD), jnp.float32),
            jax.ShapeDtypeStruct((T, WPAD), jnp.float32),
            jax.ShapeDtypeStruct((NBLK, 1), jnp.int32),
            jax.ShapeDtypeStruct((NBLK, 1), jnp.int32),
            jax.ShapeDtypeStruct((1, 1), jnp.int32),
        ],
    )(x, expert_sel)


def _dispatch_kernel(x_hbm, p0_hbm, p1_hbm, w0_hbm, w1_hbm,
                     xg_hbm, sw_hbm,
                     xrow, w0b, w1b, p0b, p1b, sem):
    wid = jax.lax.axis_index("s") * SC_CORES + jax.lax.axis_index("c")
    tbase = wid * TOK_PER_W
    pltpu.sync_copy(x_hbm.at[pl.ds(tbase, TOK_PER_W)], xrow)
    pltpu.sync_copy(w0_hbm.at[pl.ds(tbase, TOK_PER_W)], w0b)
    pltpu.sync_copy(w1_hbm.at[pl.ds(tbase, TOK_PER_W)], w1b)
    pltpu.sync_copy(p0_hbm.at[pl.ds(tbase, TOK_PER_W)], p0b)
    pltpu.sync_copy(p1_hbm.at[pl.ds(tbase, TOK_PER_W)], p1b)

    handles = []
    for ch in range(TOK_PER_W // CHUNK):
        sl = pl.ds(ch * CHUNK, CHUNK)
        idx0 = p0b[sl]
        idx1 = p1b[sl]
        handles.append(pltpu.async_copy(xrow.at[sl], xg_hbm.at[idx0], sem))
        handles.append(pltpu.async_copy(xrow.at[sl], xg_hbm.at[idx1], sem))
        handles.append(pltpu.async_copy(w0b.at[sl], sw_hbm.at[idx0], sem))
        handles.append(pltpu.async_copy(w1b.at[sl], sw_hbm.at[idx1], sem))
    for h in handles:
        h.wait()


def _run_dispatch(x, p0f, p1f, w0m, w1m):
    mesh = plsc.VectorSubcoreMesh(core_axis_name="c", subcore_axis_name="s")
    return pl.kernel(
        _dispatch_kernel,
        mesh=mesh,
        out_type=[
            jax.ShapeDtypeStruct((NSLOT, DMODEL), jnp.float32),
            jax.ShapeDtypeStruct((NSLOT, WPAD), jnp.float32),
        ],
        scratch_types=[
            pltpu.VMEM((TOK_PER_W, DMODEL), jnp.float32),
            pltpu.VMEM((TOK_PER_W, WPAD), jnp.float32),
            pltpu.VMEM((TOK_PER_W, WPAD), jnp.float32),
            pltpu.VMEM((TOK_PER_W,), jnp.int32),
            pltpu.VMEM((TOK_PER_W,), jnp.int32),
            pltpu.SemaphoreType.DMA,
        ],
    )(x, p0f, p1f, w0m, w1m)


def _expert_kernel(be_ref, oe_ref, na_ref, xg_ref, sw_ref, k_ref, v_ref,
                   yg_ref):
    i = pl.program_id(0)

    @pl.when(i < na_ref[0, 0])
    def _():
        s = jax.lax.dot_general(
            xg_ref[...], k_ref[0], (((1,), (0,)), ((), ())),
            preferred_element_type=jnp.float32)
        row = jax.lax.broadcasted_iota(jnp.int32, (BLK, 1), 0) + i * BLK
        valid = row < oe_ref[i, 0]
        wcol = sw_ref[:, 0:1]
        h = jnp.where(valid, jnp.maximum(s, 0.0) * wcol, 0.0)
        yg_ref[...] = jax.lax.dot_general(
            h, v_ref[0], (((1,), (0,)), ((), ())),
            preferred_element_type=jnp.float32)


def _run_experts(blk_e, off_end, nact, xg, slot_w, keys_w, values_w):
    def _clip(i, na):
        return jnp.minimum(i, na[0, 0] - 1)

    grid_spec = pltpu.PrefetchScalarGridSpec(
        num_scalar_prefetch=3,
        grid=(NBLK,),
        in_specs=[
            pl.BlockSpec((BLK, DMODEL),
                         lambda i, be, oe, na: (_clip(i, na), 0)),
            pl.BlockSpec((BLK, WPAD),
                         lambda i, be, oe, na: (_clip(i, na), 0)),
            pl.BlockSpec((1, DMODEL, EXPERT_SIZE),
                         lambda i, be, oe, na: (be[i, 0], 0, 0)),
            pl.BlockSpec((1, EXPERT_SIZE, DMODEL),
                         lambda i, be, oe, na: (be[i, 0], 0, 0)),
        ],
        out_specs=pl.BlockSpec((BLK, DMODEL),
                               lambda i, be, oe, na: (_clip(i, na), 0)),
    )
    return pl.pallas_call(
        _expert_kernel,
        grid_spec=grid_spec,
        out_shape=jax.ShapeDtypeStruct((NSLOT, DMODEL), jnp.float32),
    )(blk_e, off_end, nact, xg, slot_w, keys_w, values_w)


TOK_HALF = TOK_PER_W // 2


def _combine_kernel(yg_hbm, p0_hbm, p1_hbm, o0_hbm, o1_hbm,
                    buf0, buf1, p0b, p1b, sem):
    cid = jax.lax.axis_index("c")
    sid = jax.lax.axis_index("s")
    wid = sid * SC_CORES + cid
    pltpu.sync_copy(p0_hbm.at[pl.ds(wid * TOK_PER_W, TOK_PER_W)], p0b)
    pltpu.sync_copy(p1_hbm.at[pl.ds(wid * TOK_PER_W, TOK_PER_W)], p1b)

    for half in range(2):
        tbase = wid * TOK_PER_W + half * TOK_HALF
        handles = []
        for ch in range(TOK_HALF // CHUNK):
            sl = pl.ds(ch * CHUNK, CHUNK)
            psl = pl.ds(half * TOK_HALF + ch * CHUNK, CHUNK)
            handles.append(
                pltpu.async_copy(yg_hbm.at[p0b[psl]], buf0.at[sl], sem))
            handles.append(
                pltpu.async_copy(yg_hbm.at[p1b[psl]], buf1.at[sl], sem))
        for h in handles:
            h.wait()
        pltpu.sync_copy(buf0, o0_hbm.at[pl.ds(tbase, TOK_HALF)])
        pltpu.sync_copy(buf1, o1_hbm.at[pl.ds(tbase, TOK_HALF)])


def _run_combine(yg, p0f, p1f):
    mesh = plsc.VectorSubcoreMesh(core_axis_name="c", subcore_axis_name="s")
    return pl.kernel(
        _combine_kernel,
        mesh=mesh,
        out_type=[
            jax.ShapeDtypeStruct((T, DMODEL), jnp.float32),
            jax.ShapeDtypeStruct((T, DMODEL), jnp.float32),
        ],
        scratch_types=[
            pltpu.VMEM((TOK_HALF, DMODEL), jnp.float32),
            pltpu.VMEM((TOK_HALF, DMODEL), jnp.float32),
            pltpu.VMEM((TOK_PER_W,), jnp.int32),
            pltpu.VMEM((TOK_PER_W,), jnp.int32),
            pltpu.SemaphoreType.DMA,
        ],
    )(yg, p0f, p1f)


def _add_kernel(a_ref, b_ref, o_ref):
    o_ref[...] = a_ref[...] + b_ref[...]


def _run_add(o0, o1):
    nb = 8
    return pl.pallas_call(
        _add_kernel,
        grid=(nb,),
        in_specs=[
            pl.BlockSpec((T // nb, DMODEL), lambda i: (i, 0)),
            pl.BlockSpec((T // nb, DMODEL), lambda i: (i, 0)),
        ],
        out_specs=pl.BlockSpec((T // nb, DMODEL), lambda i: (i, 0)),
        out_shape=jax.ShapeDtypeStruct((T, DMODEL), jnp.float32),
    )(o0, o1)


@jax.jit
def kernel(x, expert_sel, keys_w, values_w):
    pos0, pos1, w0m, w1m, blk_e, off_end, nact = _run_router(x, expert_sel)
    p0f = pos0[:, 0]
    p1f = pos1[:, 0]
    xg, slot_w = _run_dispatch(x, p0f, p1f, w0m, w1m)
    yg = _run_experts(blk_e, off_end, nact, xg, slot_w, keys_w, values_w)
    o0, o1 = _run_combine(yg, p0f, p1f)
    return _run_add(o0, o1)


# speedup vs baseline: 1.0008x; 1.0008x over previous
import functools

import jax
import jax.numpy as jnp
from jax.experimental import pallas as pl
from jax.experimental.pallas import tpu as pltpu
from jax.experimental.pallas import tpu_sc as plsc

DMODEL = 1024
N_EXPERTS = 8
EXPERT_SIZE = 512
T = 2048

BLK = 256
NBLK = T * 2 // BLK + N_EXPERTS
NSLOT = NBLK * BLK
WPAD = 128

SC_CORES = 2
SC_SUBCORES = 16
SC_WORKERS = SC_CORES * SC_SUBCORES
TOK_PER_W = T // SC_WORKERS
CHUNK = 16


def _router_kernel(x_ref, sel_ref, pos0_ref, pos1_ref, w0_ref, w1_ref,
                   be_ref, oe_ref, na_ref):
    x = x_ref[...]
    logits = jax.lax.dot_general(
        x, sel_ref[...], (((1,), (1,)), ((), ())),
        preferred_element_type=jnp.float32)
    idx8 = jax.lax.broadcasted_iota(jnp.int32, logits.shape, 1)
    m1 = jnp.max(logits, axis=1, keepdims=True)
    i1 = jnp.min(jnp.where(logits == m1, idx8, N_EXPERTS), axis=1,
                 keepdims=True)
    oh1 = idx8 == i1
    rest = jnp.where(oh1, -jnp.inf, logits)
    m2 = jnp.max(rest, axis=1, keepdims=True)
    i2 = jnp.min(jnp.where(rest == m2, idx8, N_EXPERTS), axis=1,
                 keepdims=True)
    oh2 = idx8 == i2
    sig = jax.nn.sigmoid(logits)
    v0 = jnp.sum(jnp.where(oh1, sig, 0.0), axis=1, keepdims=True)
    v1 = jnp.sum(jnp.where(oh2, sig, 0.0), axis=1, keepdims=True)

    cnt = (oh1 | oh2).astype(jnp.float32)
    inc = cnt
    sh = 1
    while sh < T:
        inc = inc + jnp.concatenate(
            [jnp.zeros((sh, N_EXPERTS), jnp.float32), inc[:T - sh]], axis=0)
        sh *= 2
    rank = inc - cnt
    tot = inc[T - 1:T, :]
    padded = jnp.floor((tot + (BLK - 1)) / BLK) * BLK
    u_r = jax.lax.broadcasted_iota(jnp.int32, (N_EXPERTS, N_EXPERTS), 0)
    u_c = jax.lax.broadcasted_iota(jnp.int32, (N_EXPERTS, N_EXPERTS), 1)
    ustri = (u_r < u_c).astype(jnp.float32)
    off = jax.lax.dot_general(
        padded, ustri, (((1,), (0,)), ((), ())),
        preferred_element_type=jnp.float32)

    pos_base = off + rank
    p0 = jnp.sum(jnp.where(oh1, pos_base, 0.0), axis=1, keepdims=True)
    p1 = jnp.sum(jnp.where(oh2, pos_base, 0.0), axis=1, keepdims=True)

    pos0_ref[...] = jnp.broadcast_to(p0, (T, N_EXPERTS)).astype(jnp.int32)
    pos1_ref[...] = jnp.broadcast_to(p1, (T, N_EXPERTS)).astype(jnp.int32)
    w0_ref[...] = jnp.broadcast_to(v0, (T, WPAD))
    w1_ref[...] = jnp.broadcast_to(v1, (T, WPAD))

    total = jnp.sum(padded, axis=1, keepdims=True)
    na_ref[...] = (total / BLK).astype(jnp.int32)
    starts = (jax.lax.broadcasted_iota(jnp.int32, (NBLK, 1), 0) * BLK
              ).astype(jnp.float32)
    off_next = off + padded
    blk_raw = jnp.minimum(
        jnp.sum((starts >= off_next).astype(jnp.float32), axis=1,
                keepdims=True), float(N_EXPERTS - 1))
    blk_i = blk_raw.astype(jnp.int32)
    lane8 = jax.lax.broadcasted_iota(jnp.int32, (NBLK, N_EXPERTS), 1)
    onehot = blk_i == lane8
    oe_col = jnp.sum(jnp.where(onehot, off + tot, 0.0), axis=1, keepdims=True)
    be_ref[...] = blk_i
    oe_ref[...] = oe_col.astype(jnp.int32)


def _run_router(x, expert_sel):
    return pl.pallas_call(
        _router_kernel,
        out_shape=[
            jax.ShapeDtypeStruct((T, N_EXPERTS), jnp.int32),
            jax.ShapeDtypeStruct((T, N_EXPERTS), jnp.int32),
            jax.ShapeDtypeStruct((T, WPAD), jnp.float32),
            jax.ShapeDtypeStruct((T, WPAD), jnp.float32),
            jax.ShapeDtypeStruct((NBLK, 1), jnp.int32),
            jax.ShapeDtypeStruct((NBLK, 1), jnp.int32),
            jax.ShapeDtypeStruct((1, 1), jnp.int32),
        ],
    )(x, expert_sel)


def _dispatch_kernel(x_hbm, p0_hbm, p1_hbm, w0_hbm, w1_hbm,
                     xg_hbm, sw_hbm,
                     xrow, w0b, w1b, p0b, p1b, sem):
    wid = jax.lax.axis_index("s") * SC_CORES + jax.lax.axis_index("c")
    tbase = wid * TOK_PER_W
    pltpu.sync_copy(x_hbm.at[pl.ds(tbase, TOK_PER_W)], xrow)
    pltpu.sync_copy(w0_hbm.at[pl.ds(tbase, TOK_PER_W)], w0b)
    pltpu.sync_copy(w1_hbm.at[pl.ds(tbase, TOK_PER_W)], w1b)
    pltpu.sync_copy(p0_hbm.at[pl.ds(tbase, TOK_PER_W)], p0b)
    pltpu.sync_copy(p1_hbm.at[pl.ds(tbase, TOK_PER_W)], p1b)

    handles = []
    for ch in range(TOK_PER_W // CHUNK):
        sl = pl.ds(ch * CHUNK, CHUNK)
        idx0 = p0b[sl]
        idx1 = p1b[sl]
        handles.append(pltpu.async_copy(xrow.at[sl], xg_hbm.at[idx0], sem))
        handles.append(pltpu.async_copy(xrow.at[sl], xg_hbm.at[idx1], sem))
        handles.append(pltpu.async_copy(w0b.at[sl], sw_hbm.at[idx0], sem))
        handles.append(pltpu.async_copy(w1b.at[sl], sw_hbm.at[idx1], sem))
    for h in handles:
        h.wait()


def _run_dispatch(x, p0f, p1f, w0m, w1m):
    mesh = plsc.VectorSubcoreMesh(core_axis_name="c", subcore_axis_name="s")
    return pl.kernel(
        _dispatch_kernel,
        mesh=mesh,
        out_type=[
            jax.ShapeDtypeStruct((NSLOT, DMODEL), jnp.float32),
            jax.ShapeDtypeStruct((NSLOT, WPAD), jnp.float32),
        ],
        scratch_types=[
            pltpu.VMEM((TOK_PER_W, DMODEL), jnp.float32),
            pltpu.VMEM((TOK_PER_W, WPAD), jnp.float32),
            pltpu.VMEM((TOK_PER_W, WPAD), jnp.float32),
            pltpu.VMEM((TOK_PER_W,), jnp.int32),
            pltpu.VMEM((TOK_PER_W,), jnp.int32),
            pltpu.SemaphoreType.DMA,
        ],
    )(x, p0f, p1f, w0m, w1m)


def _expert_kernel(be_ref, oe_ref, na_ref, xg_ref, sw_ref, k_ref, v_ref,
                   yg_ref):
    i = pl.program_id(0)

    @pl.when(i < na_ref[0, 0])
    def _():
        e = be_ref[i, 0]
        s = jax.lax.dot_general(
            xg_ref[...], k_ref[e], (((1,), (0,)), ((), ())),
            preferred_element_type=jnp.float32)
        row = jax.lax.broadcasted_iota(jnp.int32, (BLK, 1), 0) + i * BLK
        valid = row < oe_ref[i, 0]
        wcol = sw_ref[:, 0:1]
        h = jnp.where(valid, jnp.maximum(s, 0.0) * wcol, 0.0)
        yg_ref[...] = jax.lax.dot_general(
            h, v_ref[e], (((1,), (0,)), ((), ())),
            preferred_element_type=jnp.float32)


def _run_experts(blk_e, off_end, nact, xg, slot_w, keys_w, values_w):
    def _clip(i, na):
        return jnp.minimum(i, na[0, 0] - 1)

    grid_spec = pltpu.PrefetchScalarGridSpec(
        num_scalar_prefetch=3,
        grid=(NBLK,),
        in_specs=[
            pl.BlockSpec((BLK, DMODEL),
                         lambda i, be, oe, na: (_clip(i, na), 0)),
            pl.BlockSpec((BLK, WPAD),
                         lambda i, be, oe, na: (_clip(i, na), 0)),
            pl.BlockSpec((N_EXPERTS, DMODEL, EXPERT_SIZE),
                         lambda i, be, oe, na: (0, 0, 0)),
            pl.BlockSpec((N_EXPERTS, EXPERT_SIZE, DMODEL),
                         lambda i, be, oe, na: (0, 0, 0)),
        ],
        out_specs=pl.BlockSpec((BLK, DMODEL),
                               lambda i, be, oe, na: (_clip(i, na), 0)),
    )
    return pl.pallas_call(
        _expert_kernel,
        grid_spec=grid_spec,
        out_shape=jax.ShapeDtypeStruct((NSLOT, DMODEL), jnp.float32),
    )(blk_e, off_end, nact, xg, slot_w, keys_w, values_w)


TOK_HALF = TOK_PER_W // 2


def _combine_kernel(yg_hbm, p0_hbm, p1_hbm, o0_hbm, o1_hbm,
                    buf0, buf1, p0b, p1b, sem):
    cid = jax.lax.axis_index("c")
    sid = jax.lax.axis_index("s")
    wid = sid * SC_CORES + cid
    pltpu.sync_copy(p0_hbm.at[pl.ds(wid * TOK_PER_W, TOK_PER_W)], p0b)
    pltpu.sync_copy(p1_hbm.at[pl.ds(wid * TOK_PER_W, TOK_PER_W)], p1b)

    for half in range(2):
        tbase = wid * TOK_PER_W + half * TOK_HALF
        handles = []
        for ch in range(TOK_HALF // CHUNK):
            sl = pl.ds(ch * CHUNK, CHUNK)
            psl = pl.ds(half * TOK_HALF + ch * CHUNK, CHUNK)
            handles.append(
                pltpu.async_copy(yg_hbm.at[p0b[psl]], buf0.at[sl], sem))
            handles.append(
                pltpu.async_copy(yg_hbm.at[p1b[psl]], buf1.at[sl], sem))
        for h in handles:
            h.wait()
        pltpu.sync_copy(buf0, o0_hbm.at[pl.ds(tbase, TOK_HALF)])
        pltpu.sync_copy(buf1, o1_hbm.at[pl.ds(tbase, TOK_HALF)])


def _run_combine(yg, p0f, p1f):
    mesh = plsc.VectorSubcoreMesh(core_axis_name="c", subcore_axis_name="s")
    return pl.kernel(
        _combine_kernel,
        mesh=mesh,
        out_type=[
            jax.ShapeDtypeStruct((T, DMODEL), jnp.float32),
            jax.ShapeDtypeStruct((T, DMODEL), jnp.float32),
        ],
        scratch_types=[
            pltpu.VMEM((TOK_HALF, DMODEL), jnp.float32),
            pltpu.VMEM((TOK_HALF, DMODEL), jnp.float32),
            pltpu.VMEM((TOK_PER_W,), jnp.int32),
            pltpu.VMEM((TOK_PER_W,), jnp.int32),
            pltpu.SemaphoreType.DMA,
        ],
    )(yg, p0f, p1f)


def _add_kernel(a_ref, b_ref, o_ref):
    o_ref[...] = a_ref[...] + b_ref[...]


def _run_add(o0, o1):
    nb = 8
    return pl.pallas_call(
        _add_kernel,
        grid=(nb,),
        in_specs=[
            pl.BlockSpec((T // nb, DMODEL), lambda i: (i, 0)),
            pl.BlockSpec((T // nb, DMODEL), lambda i: (i, 0)),
        ],
        out_specs=pl.BlockSpec((T // nb, DMODEL), lambda i: (i, 0)),
        out_shape=jax.ShapeDtypeStruct((T, DMODEL), jnp.float32),
    )(o0, o1)


@jax.jit
def kernel(x, expert_sel, keys_w, values_w):
    pos0, pos1, w0m, w1m, blk_e, off_end, nact = _run_router(x, expert_sel)
    p0f = pos0[:, 0]
    p1f = pos1[:, 0]
    xg, slot_w = _run_dispatch(x, p0f, p1f, w0m, w1m)
    yg = _run_experts(blk_e, off_end, nact, xg, slot_w, keys_w, values_w)
    o0, o1 = _run_combine(yg, p0f, p1f)
    return _run_add(o0, o1)


# speedup vs baseline: 1.9433x; 1.9418x over previous
import functools

import jax
import jax.numpy as jnp
from jax.experimental import pallas as pl
from jax.experimental.pallas import tpu as pltpu

DMODEL = 1024
N_EXPERTS = 8
EXPERT_SIZE = 512
N_HEADS = 2
T = 2048


def _moe_dense_kernel(x_ref, sel_ref, keys_ref, values_ref, out_ref, w_ref):
    e = pl.program_id(0)

    @pl.when(e == 0)
    def _router():
        x = x_ref[...]
        logits = jax.lax.dot_general(
            x, sel_ref[...],
            (((1,), (1,)), ((), ())),
            preferred_element_type=jnp.float32,
        )
        idx = jax.lax.broadcasted_iota(jnp.int32, logits.shape, 1)
        m1 = jnp.max(logits, axis=1, keepdims=True)
        i1 = jnp.min(jnp.where(logits == m1, idx, N_EXPERTS), axis=1, keepdims=True)
        oh1 = idx == i1
        rest = jnp.where(oh1, -jnp.inf, logits)
        m2 = jnp.max(rest, axis=1, keepdims=True)
        i2 = jnp.min(jnp.where(rest == m2, idx, N_EXPERTS), axis=1, keepdims=True)
        oh2 = idx == i2
        w_ref[...] = jax.nn.sigmoid(logits) * (oh1 | oh2).astype(jnp.float32)

    x = x_ref[...]
    scores = jax.lax.dot_general(
        x, keys_ref[0],
        (((1,), (0,)), ((), ())),
        preferred_element_type=jnp.float32,
    )
    w_all = w_ref[...]
    lane = jax.lax.broadcasted_iota(jnp.int32, w_all.shape, 1)
    w_col = jnp.sum(jnp.where(lane == e, w_all, 0.0), axis=1, keepdims=True)
    h = jnp.maximum(scores, 0.0) * w_col
    contrib = jax.lax.dot_general(
        h, values_ref[0],
        (((1,), (0,)), ((), ())),
        preferred_element_type=jnp.float32,
    )

    @pl.when(e == 0)
    def _init():
        out_ref[...] = contrib

    @pl.when(e != 0)
    def _acc():
        out_ref[...] = out_ref[...] + contrib


@jax.jit
def kernel(x, expert_sel, keys_w, values_w):
    return pl.pallas_call(
        _moe_dense_kernel,
        grid=(N_EXPERTS,),
        in_specs=[
            pl.BlockSpec((T, DMODEL), lambda e: (0, 0)),
            pl.BlockSpec((N_EXPERTS, DMODEL), lambda e: (0, 0)),
            pl.BlockSpec((1, DMODEL, EXPERT_SIZE), lambda e: (e, 0, 0)),
            pl.BlockSpec((1, EXPERT_SIZE, DMODEL), lambda e: (e, 0, 0)),
        ],
        out_specs=pl.BlockSpec((T, DMODEL), lambda e: (0, 0)),
        out_shape=jax.ShapeDtypeStruct((T, DMODEL), jnp.float32),
        scratch_shapes=[pltpu.VMEM((T, N_EXPERTS), jnp.float32)],
    )(x, expert_sel, keys_w, values_w)
